# Initial kernel scaffold; baseline (speedup 1.0000x reference)
import jax
import jax.numpy as jnp
from jax.experimental import pallas as pl


def kernel(api, seq, token, desc, emb, W1, b1, W2, b2):
    raise NotImplementedError("write your pallas kernel here")


# speedup vs baseline: 9.5645x; 9.5645x over previous
import functools

import jax
import jax.numpy as jnp
from jax import lax
from jax.experimental import pallas as pl
from jax.experimental.pallas import tpu as pltpu
from jax.experimental.pallas import tpu_sc as plsc

VOCAB = 100000
D = 128
HID = 2048
B = 4096
L = 50
NG = 4 * B
NVREG = D // 16


def _make_pool_kernel():
    info = plsc.get_sparse_core_info()
    nc, ns = info.num_cores, info.num_subcores
    nw = nc * ns
    gpw = NG // nw
    G = 8
    nchunk = gpw // G

    mesh = plsc.VectorSubcoreMesh(core_axis_name="c", subcore_axis_name="s")

    @functools.partial(
        pl.kernel,
        mesh=mesh,
        out_type=jax.ShapeDtypeStruct((NG, D), jnp.float32),
        scratch_types=[
            pltpu.VMEM((G, L), jnp.int32),
            pltpu.VMEM((G, L, D), jnp.float32),
            pltpu.VMEM((G, D), jnp.float32),
            pltpu.SemaphoreType.DMA,
        ],
    )
    def pool(emb_hbm, idx_hbm, out_hbm, idx_v, rows_v, out_v, sem):
        w = lax.axis_index("s") * nc + lax.axis_index("c")

        def chunk_body(c, carry):
            base = w * gpw + c * G
            pltpu.sync_copy(idx_hbm.at[pl.ds(base, G)], idx_v)
            copies = [
                pltpu.async_copy(emb_hbm.at[idx_v.at[g]], rows_v.at[g], sem)
                for g in range(G)
            ]
            for cp in copies:
                cp.wait()
            for g in range(G):
                def row_body(r, accs):
                    return tuple(
                        accs[v] + rows_v[g, r, pl.ds(v * 16, 16)]
                        for v in range(NVREG)
                    )
                accs = lax.fori_loop(
                    0, L, row_body,
                    tuple(jnp.zeros((16,), jnp.float32) for _ in range(NVREG)),
                )
                for v in range(NVREG):
                    out_v[g, pl.ds(v * 16, 16)] = accs[v] * (1.0 / L)
            pltpu.sync_copy(out_v, out_hbm.at[pl.ds(base, G)])
            return carry

        lax.fori_loop(0, nchunk, chunk_body, 0)

    return pool


BB = 512
NB = B // BB


def _mlp_body(a_ref, s_ref, t_ref, d_ref, w1_ref, b1_ref, w2_ref, b2_ref,
              out_ref):
    x = jnp.concatenate([a_ref[...], s_ref[...], t_ref[...]], axis=1)
    h = jnp.dot(x, w1_ref[...], preferred_element_type=jnp.float32)
    h = jnp.maximum(h + b1_ref[...], 0.0)
    y = jnp.dot(h, w2_ref[...], preferred_element_type=jnp.float32)
    r = y + b2_ref[...] - d_ref[...]
    out_ref[...] = jnp.mean(r * r, axis=1).reshape(1, BB)


def _mlp(a, s, t, d, w1, b1, w2, b2):
    pooled_spec = pl.BlockSpec((BB, D), lambda i: (i, 0))
    full = lambda shape: pl.BlockSpec(shape, lambda i: (0,) * len(shape))
    out = pl.pallas_call(
        _mlp_body,
        grid=(NB,),
        in_specs=[
            pooled_spec, pooled_spec, pooled_spec, pooled_spec,
            full((3 * D, HID)),
            full((1, HID)),
            full((HID, D)),
            full((1, D)),
        ],
        out_specs=pl.BlockSpec((1, BB), lambda i: (0, i)),
        out_shape=jax.ShapeDtypeStruct((1, B), jnp.float32),
    )(a, s, t, d, w1, b1.reshape(1, HID), w2, b2.reshape(1, D))
    return out.reshape(B)


_pool_kernel = None


def kernel(api, seq, token, desc, emb, W1, b1, W2, b2):
    global _pool_kernel
    if _pool_kernel is None:
        _pool_kernel = _make_pool_kernel()
    idx = jnp.stack([api, seq, token, desc]).reshape(NG, L).astype(jnp.int32)
    pooled = _pool_kernel(emb, idx)
    p = pooled.reshape(4, B, D)
    return _mlp(p[0], p[1], p[2], p[3], W1, b1, W2, b2)


# speedup vs baseline: 15.1927x; 1.5885x over previous
import functools

import jax
import jax.numpy as jnp
from jax import lax
from jax.experimental import pallas as pl
from jax.experimental.pallas import tpu as pltpu
from jax.experimental.pallas import tpu_sc as plsc

VOCAB = 100000
D = 128
HID = 2048
B = 4096
L = 50
NG = 4 * B
NVREG = D // 16


def _make_pool_kernel():
    info = plsc.get_sparse_core_info()
    nc, ns = info.num_cores, info.num_subcores
    nw = nc * ns
    gpw = NG // nw
    G = 8
    nchunk = gpw // G
    npair = nchunk // 2
    RU = 5

    mesh = plsc.VectorSubcoreMesh(core_axis_name="c", subcore_axis_name="s")

    @functools.partial(
        pl.kernel,
        mesh=mesh,
        out_type=jax.ShapeDtypeStruct((NG, D), jnp.float32),
        scratch_types=[
            pltpu.VMEM((G, L), jnp.int32),
            pltpu.VMEM((G, L), jnp.int32),
            pltpu.VMEM((G, L, D), jnp.float32),
            pltpu.VMEM((G, L, D), jnp.float32),
            pltpu.VMEM((G, D), jnp.float32),
            pltpu.SemaphoreType.DMA,
            pltpu.SemaphoreType.DMA,
        ],
    )
    def pool(emb_hbm, idx_hbm, out_hbm, idx0, idx1, rows0, rows1, out_v,
             sem0, sem1):
        w = lax.axis_index("s") * nc + lax.axis_index("c")
        w0 = w * gpw

        def fire(c, idx_v, rows_v, sem):
            pltpu.sync_copy(idx_hbm.at[pl.ds(w0 + c * G, G)], idx_v)
            for g in range(G):
                pltpu.async_copy(emb_hbm.at[idx_v.at[g]], rows_v.at[g], sem)

        def drain_acc_store(c, idx_v, rows_v, sem):
            for g in range(G):
                pltpu.make_async_copy(
                    emb_hbm.at[idx_v.at[g]], rows_v.at[g], sem).wait()
            for g in range(G):
                def row_body(r, accs):
                    accs = list(accs)
                    for rr in range(RU):
                        row = r * RU + rr
                        for v in range(NVREG):
                            accs[v] = accs[v] + rows_v[g, row,
                                                       pl.ds(v * 16, 16)]
                    return tuple(accs)
                accs = lax.fori_loop(
                    0, L // RU, row_body,
                    tuple(jnp.zeros((16,), jnp.float32)
                          for _ in range(NVREG)),
                )
                for v in range(NVREG):
                    out_v[g, pl.ds(v * 16, 16)] = accs[v] * (1.0 / L)
            pltpu.sync_copy(out_v, out_hbm.at[pl.ds(w0 + c * G, G)])

        fire(0, idx0, rows0, sem0)

        def pair_body(p, carry):
            c0 = 2 * p
            fire(c0 + 1, idx1, rows1, sem1)
            drain_acc_store(c0, idx0, rows0, sem0)
            fire(c0 + 2, idx0, rows0, sem0)
            drain_acc_store(c0 + 1, idx1, rows1, sem1)
            return carry

        lax.fori_loop(0, npair - 1, pair_body, 0)
        fire(nchunk - 1, idx1, rows1, sem1)
        drain_acc_store(nchunk - 2, idx0, rows0, sem0)
        drain_acc_store(nchunk - 1, idx1, rows1, sem1)

    return pool


BB = 512
NB = B // BB


def _mlp_body(a_ref, s_ref, t_ref, d_ref, w1_ref, b1_ref, w2_ref, b2_ref,
              out_ref):
    x = jnp.concatenate([a_ref[...], s_ref[...], t_ref[...]], axis=1)
    h = jnp.dot(x, w1_ref[...], preferred_element_type=jnp.float32)
    h = jnp.maximum(h + b1_ref[...], 0.0)
    y = jnp.dot(h, w2_ref[...], preferred_element_type=jnp.float32)
    r = y + b2_ref[...] - d_ref[...]
    out_ref[...] = jnp.mean(r * r, axis=1).reshape(1, BB)


def _mlp(a, s, t, d, w1, b1, w2, b2):
    pooled_spec = pl.BlockSpec((BB, D), lambda i: (i, 0))
    full = lambda shape: pl.BlockSpec(shape, lambda i: (0,) * len(shape))
    out = pl.pallas_call(
        _mlp_body,
        grid=(NB,),
        in_specs=[
            pooled_spec, pooled_spec, pooled_spec, pooled_spec,
            full((3 * D, HID)),
            full((1, HID)),
            full((HID, D)),
            full((1, D)),
        ],
        out_specs=pl.BlockSpec((1, BB), lambda i: (0, i)),
        out_shape=jax.ShapeDtypeStruct((1, B), jnp.float32),
    )(a, s, t, d, w1, b1.reshape(1, HID), w2, b2.reshape(1, D))
    return out.reshape(B)


_pool_kernel = None


def kernel(api, seq, token, desc, emb, W1, b1, W2, b2):
    global _pool_kernel
    if _pool_kernel is None:
        _pool_kernel = _make_pool_kernel()
    idx = jnp.stack([api, seq, token, desc]).reshape(NG, L).astype(jnp.int32)
    pooled = _pool_kernel(emb, idx)
    p = pooled.reshape(4, B, D)
    return _mlp(p[0], p[1], p[2], p[3], W1, b1, W2, b2)
